# baseline (speedup 1.0000x reference)
import functools

import jax
import jax.numpy as jnp
from jax import lax
from jax.experimental import pallas as pl
from jax.experimental.pallas import tpu as pltpu
from jax.experimental.pallas import tpu_sc as plsc

B, T, S, D = 4, 2048, 2048, 1024
_L = 16


def _sc_cols_phase(inp_hbm, idx_hbm, toks_v, cols_v, cid, sid):
    @pl.when(sid < 2)
    def _():
        b = cid + 2 * sid
        pltpu.sync_copy(inp_hbm.at[pl.ds(b * T, T)], toks_v)

        def step(k, carry):
            x = toks_v[pl.ds(k * _L, _L)]
            m = jnp.minimum(jnp.maximum(3 - x, 0), 1)
            gidx = lax.iota(jnp.int32, _L) + k * _L
            m = m * jnp.minimum(gidx, 1)
            c = plsc.cumsum(m) + carry
            col = gidx - c - 1
            sign = lax.shift_right_arithmetic(col, 31)
            col = col - sign * S
            col = col * jnp.minimum(gidx, 1)
            cols_v[pl.ds(k * _L, _L)] = col + b * S
            return jnp.max(c)

        lax.fori_loop(0, T // _L, step, jnp.int32(0))
        pltpu.sync_copy(cols_v, idx_hbm.at[pl.ds(b * T, T)])


_GCH = 32


def _sc_colgather_body(inp_hbm, ctx_hbm, idx_hbm, mix_hbm, toks_v, cols_v,
                       idx_v, rows_v0, rows_v1, g0, g1, o0, o1):
    cid = lax.axis_index("c")
    sid = lax.axis_index("s")
    _sc_cols_phase(inp_hbm, idx_hbm, toks_v, cols_v, cid, sid)
    plsc.subcore_barrier()
    rows_per_w = (B * T) // 32
    nch = rows_per_w // _GCH
    base = jnp.where(sid < 8,
                     cid * T + sid * rows_per_w,
                     (cid + 2) * T + (sid - 8) * rows_per_w)
    pltpu.sync_copy(idx_hbm.at[pl.ds(base, rows_per_w)], idx_v)
    bufs = (rows_v0, rows_v1)
    gsem = (g0, g1)
    osem = (o0, o1)
    gather_cp = [None, None]
    out_cp = [None, None]

    def start_gather(ch):
        buf = ch % 2
        gather_cp[buf] = pltpu.async_copy(
            ctx_hbm.at[idx_v.at[pl.ds(ch * _GCH, _GCH)]], bufs[buf], gsem[buf])

    start_gather(0)
    for ch in range(nch):
        buf = ch % 2
        if ch + 1 < nch:
            if out_cp[1 - buf] is not None:
                out_cp[1 - buf].wait()
                out_cp[1 - buf] = None
            start_gather(ch + 1)
        gather_cp[buf].wait()
        out_cp[buf] = pltpu.async_copy(
            bufs[buf], mix_hbm.at[pl.ds(base + ch * _GCH, _GCH)], osem[buf])
    for buf in range(2):
        if out_cp[buf] is not None:
            out_cp[buf].wait()


def _sc_colgather(inp_flat, ctx2d):
    mesh = plsc.VectorSubcoreMesh(core_axis_name="c", subcore_axis_name="s")
    fn = functools.partial(
        pl.kernel,
        mesh=mesh,
        out_type=(
            jax.ShapeDtypeStruct((B * T,), jnp.int32),
            jax.ShapeDtypeStruct((B * T, D), jnp.float32),
        ),
        scratch_types=[
            pltpu.VMEM((T,), jnp.int32),
            pltpu.VMEM((T,), jnp.int32),
            pltpu.VMEM(((B * T) // 32,), jnp.int32),
            pltpu.VMEM((_GCH, D), jnp.float32),
            pltpu.VMEM((_GCH, D), jnp.float32),
            pltpu.SemaphoreType.DMA,
            pltpu.SemaphoreType.DMA,
            pltpu.SemaphoreType.DMA,
            pltpu.SemaphoreType.DMA,
        ],
        compiler_params=pltpu.CompilerParams(needs_layout_passes=False),
    )(_sc_colgather_body)
    return fn(inp_flat, ctx2d)


_ZR = 16


def _sc_attn_body(idx_hbm, zeros_hbm, mix_hbm, attn_hbm, zb0, zb1, idx_v,
                  sem0, sem1):
    del mix_hbm
    cid = lax.axis_index("c")
    sid = lax.axis_index("s")
    wid = sid * 2 + cid
    rows_per_w = (B * T) // 32
    row0 = wid * rows_per_w

    pltpu.sync_copy(zeros_hbm, zb0)
    pltpu.sync_copy(zeros_hbm, zb1)
    pltpu.sync_copy(idx_hbm.at[pl.ds(row0, rows_per_w)], idx_v)

    bufs = (zb0, zb1)
    sems = (sem0, sem1)
    cps = [None, None]
    prev_col = [None, None]
    riota = lax.iota(jnp.int32, _L)
    ones = jnp.full((_L,), 1.0, jnp.float32)
    zeros = jnp.zeros((_L,), jnp.float32)
    nv = _ZR // _L
    for c in range(rows_per_w // _ZR):
        bf = c % 2
        if cps[bf] is not None:
            cps[bf].wait()
            for v in range(nv):
                plsc.store_scatter(bufs[bf], [v * _L + riota, prev_col[bf][v]],
                                   zeros)
        cols = []
        for v in range(nv):
            flat = idx_v[pl.ds(c * _ZR + v * _L, _L)]
            r = riota + (row0 + c * _ZR + v * _L)
            col = flat - lax.shift_right_logical(r, 11) * S
            plsc.store_scatter(bufs[bf], [v * _L + riota, col], ones)
            cols.append(col)
        cps[bf] = pltpu.async_copy(
            bufs[bf], attn_hbm.at[pl.ds(row0 + c * _ZR, _ZR)], sems[bf])
        prev_col[bf] = cols
    cps[0].wait()
    cps[1].wait()


def _sc_attn(idx_flat, zeros_hbm, mix):
    mesh = plsc.VectorSubcoreMesh(core_axis_name="c", subcore_axis_name="s")
    fn = functools.partial(
        pl.kernel,
        mesh=mesh,
        out_type=jax.ShapeDtypeStruct((B * T, S), jnp.float32),
        scratch_types=[
            pltpu.VMEM((_ZR, S), jnp.float32),
            pltpu.VMEM((_ZR, S), jnp.float32),
            pltpu.VMEM(((B * T) // 32,), jnp.int32),
            pltpu.SemaphoreType.DMA,
            pltpu.SemaphoreType.DMA,
        ],
        compiler_params=pltpu.CompilerParams(needs_layout_passes=False),
    )(_sc_attn_body)
    return fn(idx_flat, zeros_hbm, mix)


_BR = 1024


def _out_body(mix_ref, y_ref, w_ref, b_ref, o_ref):
    w1 = w_ref[:, :D]
    w2 = w_ref[:, D:]
    acc = lax.dot_general(
        mix_ref[...].astype(jnp.bfloat16), w1, (((1,), (1,)), ((), ())),
        preferred_element_type=jnp.float32,
    )
    acc = acc + lax.dot_general(
        y_ref[...].astype(jnp.bfloat16), w2, (((1,), (1,)), ((), ())),
        preferred_element_type=jnp.float32,
    )
    o_ref[...] = jnp.tanh(acc + b_ref[...])


def _tc_out(mix, y2d, W, b2d):
    return pl.pallas_call(
        _out_body,
        grid=((B * T) // _BR,),
        in_specs=[
            pl.BlockSpec((_BR, D), lambda r: (r, 0)),
            pl.BlockSpec((_BR, D), lambda r: (r, 0)),
            pl.BlockSpec((D, 2 * D), lambda r: (0, 0)),
            pl.BlockSpec((1, D), lambda r: (0, 0)),
        ],
        out_specs=pl.BlockSpec((_BR, D), lambda r: (r, 0)),
        out_shape=jax.ShapeDtypeStruct((B * T, D), jnp.float32),
    )(mix, y2d, W.astype(jnp.bfloat16), b2d)


def kernel(input_var, output, context, di, W, b):
    del di
    inp_flat = input_var.reshape(B * T)
    idx_flat, mix = _sc_colgather(inp_flat, context.reshape(B * S, D))
    zeros_src = jnp.zeros((_ZR, S), jnp.float32)
    attn = _sc_attn(idx_flat, zeros_src, mix)
    out = _tc_out(mix, output.reshape(B * T, D), W, b.reshape(1, D))
    return out.reshape(B, T, D), attn.reshape(B, T, S)


# speedup vs baseline: 1.0422x; 1.0422x over previous
import functools

import jax
import jax.numpy as jnp
from jax import lax
from jax.experimental import pallas as pl
from jax.experimental.pallas import tpu as pltpu
from jax.experimental.pallas import tpu_sc as plsc

B, T, S, D = 4, 2048, 2048, 1024
_L = 16


def _sc_cols_phase(inp_hbm, idx_hbm, toks_v, cols_v, cid, sid):
    @pl.when(sid < 2)
    def _():
        b = cid + 2 * sid
        pltpu.sync_copy(inp_hbm.at[pl.ds(b * T, T)], toks_v)

        def step(k, carry):
            x = toks_v[pl.ds(k * _L, _L)]
            m = jnp.minimum(jnp.maximum(3 - x, 0), 1)
            gidx = lax.iota(jnp.int32, _L) + k * _L
            m = m * jnp.minimum(gidx, 1)
            c = plsc.cumsum(m) + carry
            col = gidx - c - 1
            sign = lax.shift_right_arithmetic(col, 31)
            col = col - sign * S
            col = col * jnp.minimum(gidx, 1)
            cols_v[pl.ds(k * _L, _L)] = col + b * S
            return jnp.max(c)

        lax.fori_loop(0, T // _L, step, jnp.int32(0))
        pltpu.sync_copy(cols_v, idx_hbm.at[pl.ds(b * T, T)])


_GCH = 32


def _sc_colgather_body(inp_hbm, ctx_hbm, idx_hbm, mix_hbm, toks_v, cols_v,
                       idx_v, rows_v0, rows_v1, g0, g1, o0, o1):
    cid = lax.axis_index("c")
    sid = lax.axis_index("s")
    _sc_cols_phase(inp_hbm, idx_hbm, toks_v, cols_v, cid, sid)
    plsc.subcore_barrier()
    rows_per_w = (B * T) // 32
    nch = rows_per_w // _GCH
    base = jnp.where(sid < 8,
                     cid * T + sid * rows_per_w,
                     (cid + 2) * T + (sid - 8) * rows_per_w)
    pltpu.sync_copy(idx_hbm.at[pl.ds(base, rows_per_w)], idx_v)
    bufs = (rows_v0, rows_v1)
    gsem = (g0, g1)
    osem = (o0, o1)
    gather_cp = [None, None]
    out_cp = [None, None]

    def start_gather(ch):
        buf = ch % 2
        gather_cp[buf] = pltpu.async_copy(
            ctx_hbm.at[idx_v.at[pl.ds(ch * _GCH, _GCH)]], bufs[buf], gsem[buf])

    start_gather(0)
    for ch in range(nch):
        buf = ch % 2
        if ch + 1 < nch:
            if out_cp[1 - buf] is not None:
                out_cp[1 - buf].wait()
                out_cp[1 - buf] = None
            start_gather(ch + 1)
        gather_cp[buf].wait()
        out_cp[buf] = pltpu.async_copy(
            bufs[buf], mix_hbm.at[pl.ds(base + ch * _GCH, _GCH)], osem[buf])
    for buf in range(2):
        if out_cp[buf] is not None:
            out_cp[buf].wait()


def _sc_colgather(inp_flat, ctx2d):
    mesh = plsc.VectorSubcoreMesh(core_axis_name="c", subcore_axis_name="s")
    fn = functools.partial(
        pl.kernel,
        mesh=mesh,
        out_type=(
            jax.ShapeDtypeStruct((B * T,), jnp.int32),
            jax.ShapeDtypeStruct((B * T, D), jnp.float32),
        ),
        scratch_types=[
            pltpu.VMEM((T,), jnp.int32),
            pltpu.VMEM((T,), jnp.int32),
            pltpu.VMEM(((B * T) // 32,), jnp.int32),
            pltpu.VMEM((_GCH, D), jnp.float32),
            pltpu.VMEM((_GCH, D), jnp.float32),
            pltpu.SemaphoreType.DMA,
            pltpu.SemaphoreType.DMA,
            pltpu.SemaphoreType.DMA,
            pltpu.SemaphoreType.DMA,
        ],
        compiler_params=pltpu.CompilerParams(needs_layout_passes=False),
    )(_sc_colgather_body)
    return fn(inp_flat, ctx2d)


_ZR = 16


def _sc_attn_body(idx_hbm, zeros_hbm, mix_hbm, attn_hbm, zb0, zb1, idx_v,
                  sem0, sem1):
    del mix_hbm
    cid = lax.axis_index("c")
    sid = lax.axis_index("s")
    wid = sid * 2 + cid
    rows_per_w = (B * T) // 32
    row0 = wid * rows_per_w

    pltpu.sync_copy(zeros_hbm, zb0)
    pltpu.sync_copy(zeros_hbm, zb1)
    pltpu.sync_copy(idx_hbm.at[pl.ds(row0, rows_per_w)], idx_v)

    bufs = (zb0, zb1)
    sems = (sem0, sem1)
    cps = [None, None]
    prev_col = [None, None]
    riota = lax.iota(jnp.int32, _L)
    ones = jnp.full((_L,), 1.0, jnp.float32)
    zeros = jnp.zeros((_L,), jnp.float32)
    nv = _ZR // _L
    for c in range(rows_per_w // _ZR):
        bf = c % 2
        if cps[bf] is not None:
            cps[bf].wait()
            for v in range(nv):
                plsc.store_scatter(bufs[bf], [v * _L + riota, prev_col[bf][v]],
                                   zeros)
        cols = []
        for v in range(nv):
            flat = idx_v[pl.ds(c * _ZR + v * _L, _L)]
            r = riota + (row0 + c * _ZR + v * _L)
            col = flat - lax.shift_right_logical(r, 11) * S
            plsc.store_scatter(bufs[bf], [v * _L + riota, col], ones)
            cols.append(col)
        cps[bf] = pltpu.async_copy(
            bufs[bf], attn_hbm.at[pl.ds(row0 + c * _ZR, _ZR)], sems[bf])
        prev_col[bf] = cols
    cps[0].wait()
    cps[1].wait()


def _sc_attn(idx_flat, zeros_hbm, mix):
    mesh = plsc.VectorSubcoreMesh(core_axis_name="c", subcore_axis_name="s")
    fn = functools.partial(
        pl.kernel,
        mesh=mesh,
        out_type=jax.ShapeDtypeStruct((B * T, S), jnp.float32),
        scratch_types=[
            pltpu.VMEM((_ZR, S), jnp.float32),
            pltpu.VMEM((_ZR, S), jnp.float32),
            pltpu.VMEM(((B * T) // 32,), jnp.int32),
            pltpu.SemaphoreType.DMA,
            pltpu.SemaphoreType.DMA,
        ],
        compiler_params=pltpu.CompilerParams(needs_layout_passes=False),
    )(_sc_attn_body)
    return fn(idx_flat, zeros_hbm, mix)


_BR = 1024


def _out_body(mix_ref, y_ref, w_ref, b_ref, o_ref):
    w1 = w_ref[:, :D]
    w2 = w_ref[:, D:]
    acc = lax.dot_general(
        mix_ref[...].astype(jnp.bfloat16), w1, (((1,), (1,)), ((), ())),
        preferred_element_type=jnp.float32,
    )
    acc = acc + lax.dot_general(
        y_ref[...].astype(jnp.bfloat16), w2, (((1,), (1,)), ((), ())),
        preferred_element_type=jnp.float32,
    )
    o_ref[...] = jnp.tanh(acc + b_ref[...])


def _tc_out(mix, y2d, W, b2d):
    return pl.pallas_call(
        _out_body,
        grid=((B * T) // _BR,),
        in_specs=[
            pl.BlockSpec((_BR, D), lambda r: (r, 0)),
            pl.BlockSpec((_BR, D), lambda r: (r, 0)),
            pl.BlockSpec((D, 2 * D), lambda r: (0, 0)),
            pl.BlockSpec((1, D), lambda r: (0, 0)),
        ],
        out_specs=pl.BlockSpec((_BR, D), lambda r: (r, 0)),
        out_shape=jax.ShapeDtypeStruct((B * T, D), jnp.float32),
    )(mix, y2d, W, b2d)


def kernel(input_var, output, context, di, W, b):
    del di
    inp_flat = input_var.reshape(B * T)
    idx_flat, mix = _sc_colgather(inp_flat, context.reshape(B * S, D))
    zeros_src = jnp.zeros((_ZR, S), jnp.float32)
    attn = _sc_attn(idx_flat, zeros_src, mix)
    out = _tc_out(mix, output.reshape(B * T, D), W, b.reshape(1, D))
    return out.reshape(B, T, D), attn.reshape(B, T, S)
